# baseline (speedup 1.0000x reference)
import functools

import jax
import jax.numpy as jnp
from jax import lax
from jax.experimental import pallas as pl
from jax.experimental.pallas import tpu as pltpu
from jax.experimental.pallas import tpu_sc as plsc

CH = 1664


def kernel(user_emb, item_emb):
    ut = user_emb.T
    it = item_emb.T
    d, n = ut.shape
    info = plsc.get_sparse_core_info()
    nw = info.num_cores * info.num_subcores
    nc = info.num_cores

    nfull = n // CH
    cpw = nfull // nw
    main_cols = cpw * nw * CH
    leftover = n - main_cols
    tail_ch = (leftover // nw) // 128 * 128
    rem2 = leftover - tail_ch * nw

    mesh = plsc.VectorSubcoreMesh(core_axis_name="c", subcore_axis_name="s")

    scratch = [
        pltpu.VMEM((d, CH), jnp.float32),
        pltpu.VMEM((d, CH), jnp.float32),
        pltpu.SemaphoreType.DMA((2,)),
        pltpu.SemaphoreType.DMA((2,)),
    ]
    if rem2:
        scratch.append(pltpu.VMEM((d, rem2), jnp.float32))

    @functools.partial(
        pl.kernel,
        mesh=mesh,
        out_type=(
            jax.ShapeDtypeStruct(ut.shape, ut.dtype),
            jax.ShapeDtypeStruct(it.shape, it.dtype),
        ),
        scratch_types=scratch,
    )
    def sc_copy(u_in, i_in, u_out, i_out, buf0, buf1, rsem, wsem, *rembuf):
        wid = lax.axis_index("s") * nc + lax.axis_index("c")
        bufs = (buf0, buf1)

        def copy_table(src, dst):
            def rd(k):
                off = (wid * cpw + k) * CH
                return pltpu.make_async_copy(
                    src.at[:, pl.ds(off, CH)], bufs[k % 2], rsem.at[k % 2]
                )

            def wr(k):
                off = (wid * cpw + k) * CH
                return pltpu.make_async_copy(
                    bufs[k % 2], dst.at[:, pl.ds(off, CH)], wsem.at[k % 2]
                )

            rd(0).start()
            for k in range(cpw):
                if k + 1 < cpw:
                    if k >= 1:
                        wr(k - 1).wait()
                    rd(k + 1).start()
                rd(k).wait()
                wr(k).start()
            if cpw >= 2:
                wr(cpw - 2).wait()
            wr(cpw - 1).wait()

            if tail_ch:
                off = main_cols + wid * tail_ch
                tbuf = buf0.at[:, : tail_ch]
                pltpu.sync_copy(src.at[:, pl.ds(off, tail_ch)], tbuf)
                pltpu.sync_copy(tbuf, dst.at[:, pl.ds(off, tail_ch)])
            if rem2:

                @pl.when(wid == 0)
                def _():
                    off = n - rem2
                    pltpu.sync_copy(src.at[:, pl.ds(off, rem2)], rembuf[0])
                    pltpu.sync_copy(rembuf[0], dst.at[:, pl.ds(off, rem2)])

        copy_table(u_in, u_out)
        copy_table(i_in, i_out)

    out_ut, out_it = sc_copy(ut, it)
    return out_ut.T, out_it.T
